# baseline (speedup 1.0000x reference)
import jax
import jax.numpy as jnp
from jax import lax
from jax.experimental import pallas as pl
from jax.experimental.pallas import tpu as pltpu
from jax.experimental.pallas import tpu_sc as plsc

NUM_TYPES = 512
HIDDEN = 256
EDGES = 160000

NC = 2
NS = 16
NPAIR = NC * NS // 2
PAIR_EDGES = EDGES // NPAIR
HCOL = HIDDEN // 2
CHUNK = 96
NFULL = PAIR_EDGES // CHUNK
TAIL = PAIR_EDGES - NFULL * CHUNK
SC_EDGES = EDGES // NC
TPS = SC_EDGES // NS

assert NFULL % 2 == 0 and CHUNK % 8 == 0 and TAIL % 8 == 0


def _body(table_hbm, idx_hbm, out_hbm, tbl_v, buf, idx_stage, idx_sp, idx_sm,
          tail_sm, ssem0, ssem1, tsem, isem):
    c = lax.axis_index("c")
    s = lax.axis_index("s")
    pair = s // 2
    h = s % 2
    col0 = pl.multiple_of(h * HCOL, 8)
    pair_base = c * SC_EDGES + pair * PAIR_EDGES
    sp_base = pair * PAIR_EDGES
    ssems = (ssem0, ssem1)

    tcopy = pltpu.async_copy(table_hbm.at[:, pl.ds(col0, HCOL)], tbl_v, tsem)
    pltpu.async_copy(idx_hbm.at[pl.ds(pair_base, PAIR_EDGES)], idx_stage,
                     isem).wait()
    my_sp = pl.multiple_of(s * PAIR_EDGES, 8)
    pltpu.async_copy(idx_stage, idx_sp.at[pl.ds(my_sp, PAIR_EDGES)],
                     isem).wait()

    def swait(b):
        pltpu.make_async_copy(
            buf.at[b], out_hbm.at[pl.ds(0, CHUNK), pl.ds(0, HCOL)],
            ssems[b]).wait()

    def do_chunk(cidx, b, wait_store):
        off = pl.multiple_of(cidx * CHUNK, 8)
        pltpu.sync_copy(idx_sp.at[pl.ds(my_sp + off, CHUNK)], idx_sm.at[b])
        if wait_store:
            swait(b)

        @plsc.parallel_loop(0, CHUNK, step=1, unroll=4)
        def _(j):
            t = idx_sm[b, j]
            for v in range(HCOL // 16):
                buf[b, j, pl.ds(16 * v, 16)] = tbl_v[t, pl.ds(16 * v, 16)]

        pltpu.async_copy(
            buf.at[b],
            out_hbm.at[pl.ds(pair_base + off, CHUNK), pl.ds(col0, HCOL)],
            ssems[b])

    tcopy.wait()

    do_chunk(0, 0, False)
    do_chunk(1, 1, False)

    def two(t, carry):
        for b in range(2):
            do_chunk(2 * t + b, b, True)
        return carry

    lax.fori_loop(1, NFULL // 2, two, 0)

    swait(0)
    swait(1)
    toff = NFULL * CHUNK
    pltpu.sync_copy(idx_sp.at[pl.ds(my_sp + toff, TAIL)], tail_sm)

    @plsc.parallel_loop(0, TAIL, step=1, unroll=2)
    def _(j):
        t = tail_sm[j]
        for v in range(HCOL // 16):
            buf[0, j, pl.ds(16 * v, 16)] = tbl_v[t, pl.ds(16 * v, 16)]

    pltpu.sync_copy(
        buf.at[0, pl.ds(0, TAIL)],
        out_hbm.at[pl.ds(pair_base + toff, TAIL), pl.ds(col0, HCOL)])


def _build():
    mesh = plsc.VectorSubcoreMesh(
        core_axis_name="c", subcore_axis_name="s", num_cores=NC,
        num_subcores=NS)
    return pl.kernel(
        _body,
        out_type=jax.ShapeDtypeStruct((EDGES, HIDDEN), jnp.float32),
        mesh=mesh,
        scratch_types=[
            pltpu.VMEM((NUM_TYPES, HCOL), jnp.float32),
            pltpu.VMEM((2, CHUNK, HCOL), jnp.float32),
            pltpu.VMEM((PAIR_EDGES,), jnp.int32),
            pltpu.VMEM_SHARED((NS * PAIR_EDGES,), jnp.int32),
            pltpu.SMEM((2, CHUNK), jnp.int32),
            pltpu.SMEM((TAIL,), jnp.int32),
            pltpu.SemaphoreType.DMA,
            pltpu.SemaphoreType.DMA,
            pltpu.SemaphoreType.DMA,
            pltpu.SemaphoreType.DMA,
        ],
    )


def kernel(type_indices, type_embedding):
    idx = type_indices.astype(jnp.int32)
    return _build()(type_embedding, idx)


# speedup vs baseline: 1.1030x; 1.1030x over previous
import jax
import jax.numpy as jnp
from jax import lax
from jax.experimental import pallas as pl
from jax.experimental.pallas import tpu as pltpu
from jax.experimental.pallas import tpu_sc as plsc

NUM_TYPES = 512
HIDDEN = 256
EDGES = 160000

NC = 2
NS = 16
NPAIR = NC * NS // 2
PAIR_EDGES = EDGES // NPAIR
HCOL = HIDDEN // 2
CHUNK = 96
NFULL = PAIR_EDGES // CHUNK
TAIL = PAIR_EDGES - NFULL * CHUNK
SC_EDGES = EDGES // NC
TPS = SC_EDGES // NS

assert NFULL % 2 == 0 and CHUNK % 8 == 0 and TAIL % 8 == 0


def _body(table_hbm, idx_hbm, out_hbm, tbl_v, buf, idx_stage, idx_sp, idx_sm,
          tail_sm, ssem0, ssem1, tsem, isem):
    c = lax.axis_index("c")
    s = lax.axis_index("s")
    pair = s // 2
    h = s % 2
    col0 = pl.multiple_of(h * HCOL, 8)
    pair_base = c * SC_EDGES + pair * PAIR_EDGES
    sp_base = pair * PAIR_EDGES
    ssems = (ssem0, ssem1)

    tcopy = pltpu.async_copy(table_hbm.at[:, pl.ds(col0, HCOL)], tbl_v, tsem)
    pltpu.async_copy(idx_hbm.at[pl.ds(pair_base, PAIR_EDGES)], idx_stage,
                     isem).wait()
    my_sp = pl.multiple_of(s * PAIR_EDGES, 8)
    pltpu.async_copy(idx_stage, idx_sp.at[pl.ds(my_sp, PAIR_EDGES)],
                     isem).wait()

    def swait(b):
        pltpu.make_async_copy(
            buf.at[b], out_hbm.at[pl.ds(0, CHUNK), pl.ds(0, HCOL)],
            ssems[b]).wait()

    def ipf(cidx, b):
        off = pl.multiple_of(
            jnp.minimum(cidx, NFULL - 1) * CHUNK, 8)
        pltpu.async_copy(idx_sp.at[pl.ds(my_sp + off, CHUNK)], idx_sm.at[b],
                         isem)

    def iwait(b):
        pltpu.make_async_copy(idx_sp.at[pl.ds(my_sp, CHUNK)], idx_sm.at[b],
                              isem).wait()

    def do_chunk(cidx, b, wait_store):
        off = pl.multiple_of(cidx * CHUNK, 8)
        iwait(b)
        ipf(cidx + 1, 1 - b)
        if wait_store:
            swait(b)

        @plsc.parallel_loop(0, CHUNK, step=1, unroll=4)
        def _(j):
            t = idx_sm[b, j]
            for v in range(HCOL // 16):
                buf[b, j, pl.ds(16 * v, 16)] = tbl_v[t, pl.ds(16 * v, 16)]

        pltpu.async_copy(
            buf.at[b],
            out_hbm.at[pl.ds(pair_base + off, CHUNK), pl.ds(col0, HCOL)],
            ssems[b])

    ipf(0, 0)
    tcopy.wait()

    do_chunk(0, 0, False)
    do_chunk(1, 1, False)

    def two(t, carry):
        for b in range(2):
            do_chunk(2 * t + b, b, True)
        return carry

    lax.fori_loop(1, NFULL // 2, two, 0)

    iwait(0)
    swait(0)
    swait(1)
    toff = NFULL * CHUNK
    pltpu.sync_copy(idx_sp.at[pl.ds(my_sp + toff, TAIL)], tail_sm)

    @plsc.parallel_loop(0, TAIL, step=1, unroll=2)
    def _(j):
        t = tail_sm[j]
        for v in range(HCOL // 16):
            buf[0, j, pl.ds(16 * v, 16)] = tbl_v[t, pl.ds(16 * v, 16)]

    pltpu.sync_copy(
        buf.at[0, pl.ds(0, TAIL)],
        out_hbm.at[pl.ds(pair_base + toff, TAIL), pl.ds(col0, HCOL)])


def _build():
    mesh = plsc.VectorSubcoreMesh(
        core_axis_name="c", subcore_axis_name="s", num_cores=NC,
        num_subcores=NS)
    return pl.kernel(
        _body,
        out_type=jax.ShapeDtypeStruct((EDGES, HIDDEN), jnp.float32),
        mesh=mesh,
        scratch_types=[
            pltpu.VMEM((NUM_TYPES, HCOL), jnp.float32),
            pltpu.VMEM((2, CHUNK, HCOL), jnp.float32),
            pltpu.VMEM((PAIR_EDGES,), jnp.int32),
            pltpu.VMEM_SHARED((NS * PAIR_EDGES,), jnp.int32),
            pltpu.SMEM((2, CHUNK), jnp.int32),
            pltpu.SMEM((TAIL,), jnp.int32),
            pltpu.SemaphoreType.DMA,
            pltpu.SemaphoreType.DMA,
            pltpu.SemaphoreType.DMA,
            pltpu.SemaphoreType.DMA,
        ],
    )


def kernel(type_indices, type_embedding):
    idx = type_indices.astype(jnp.int32)
    return _build()(type_embedding, idx)
